# baseline (speedup 1.0000x reference)
import functools

import jax
import jax.numpy as jnp
from jax import lax
from jax.experimental import pallas as pl
from jax.experimental.pallas import tpu as pltpu
from jax.experimental.pallas import tpu_sc as plsc

NC = 2
NS = 16
NW = NC * NS
LANES = 16
SEQ_PW = 32
PBLK = 4
EPS = 1e-5

_DNUMS = lax.GatherDimensionNumbers(
    offset_dims=(), collapsed_slice_dims=(0,), start_index_map=(0,))


def _permute(v, perm):
    return lax.gather(v, perm.reshape(LANES, 1), _DNUMS, (1,),
                      mode=lax.GatherScatterMode.PROMISE_IN_BOUNDS)


def _lane_sum(v):
    for k in (1, 2, 4, 8):
        v = v + _permute(v, lax.iota(jnp.int32, LANES) ^ k)
    return v


def _rsqrt(x):
    i = lax.bitcast_convert_type(x, jnp.int32)
    i = jnp.int32(0x5F3759DF) - lax.shift_right_logical(i, 1)
    y = lax.bitcast_convert_type(i, jnp.float32)
    for _ in range(3):
        y = y * (1.5 - 0.5 * x * y * y)
    return y


def _make_sc_kernel(n_rows, seq_len, hidden):
    ch = PBLK * SEQ_PW
    nch = seq_len // PBLK
    nvec = hidden // LANES
    srw = PBLK * hidden
    nsr = n_rows // PBLK
    mesh = plsc.VectorSubcoreMesh(
        core_axis_name="c", subcore_axis_name="s",
        num_cores=NC, num_subcores=NS)

    @functools.partial(
        pl.kernel,
        out_type=jax.ShapeDtypeStruct((nsr, srw), jnp.float32),
        mesh=mesh,
        scratch_types=[
            pltpu.VMEM((nch, ch), jnp.int32),
            pltpu.VMEM((nch, ch), jnp.int32),
            pltpu.VMEM((2, SEQ_PW), jnp.int32),
            pltpu.VMEM((seq_len, hidden), jnp.float32),
            pltpu.VMEM((hidden,), jnp.float32),
            pltpu.VMEM((2, ch, hidden), jnp.float32),
            pltpu.VMEM((2, SEQ_PW, srw), jnp.float32),
            pltpu.VMEM((2, hidden), jnp.float32),
            pltpu.VMEM((PBLK * SEQ_PW * LANES,), jnp.float32),
            pltpu.SemaphoreType.DMA,
            pltpu.SemaphoreType.DMA,
        ],
    )
    def sc_kernel(tok_hbm, seg_hbm, word_hbm, pos_hbm, dt_hbm, gb_hbm,
                  out_hbm, tok_v, seg_v, oidx_v, pos_v, dt_v, wbuf, obuf,
                  gb_v, segf_v, sem_w, sem_o):
        wid = lax.axis_index("s") * NC + lax.axis_index("c")
        pltpu.sync_copy(tok_hbm.at[wid], tok_v)
        pltpu.sync_copy(seg_hbm.at[wid], seg_v)
        pltpu.sync_copy(gb_hbm, gb_v)
        pltpu.sync_copy(pos_hbm, pos_v)
        pltpu.sync_copy(dt_hbm, dt_v)

        g = [gb_v[0, pl.ds(LANES * j, LANES)] for j in range(nvec)]
        bta = [gb_v[1, pl.ds(LANES * j, LANES)] for j in range(nvec)]
        dt = [dt_v[pl.ds(LANES * j, LANES)] for j in range(nvec)]

        def issue_gather(c, slot):
            pltpu.async_copy(word_hbm.at[tok_v.at[c]], wbuf.at[slot], sem_w)

        def wait_gather(c, slot):
            pltpu.make_async_copy(word_hbm.at[tok_v.at[c]],
                                  wbuf.at[slot], sem_w).wait()

        def issue_scatter(slot):
            pltpu.async_copy(obuf.at[slot], out_hbm.at[oidx_v.at[slot]],
                             sem_o)

        def wait_scatter(slot):
            pltpu.make_async_copy(obuf.at[slot],
                                  out_hbm.at[oidx_v.at[slot]], sem_o).wait()

        issue_gather(0, 0)
        issue_gather(1, 1)

        def row_norm(slot, p, orow, wrow, pt0):
            segf = segf_v[pl.ds(LANES * wrow, LANES)]
            x = [wbuf[slot, wrow, pl.ds(LANES * j, LANES)]
                 + (segf * dt[j] + pt0[j])
                 for j in range(nvec)]
            s = x[0]
            sq = x[0] * x[0]
            for j in range(1, nvec):
                s = s + x[j]
                sq = sq + x[j] * x[j]
            mean = _lane_sum(s) * (1.0 / hidden)
            ex2 = _lane_sum(sq) * (1.0 / hidden)
            var = ex2 - mean * mean
            rstd = _rsqrt(var + EPS)
            c0 = -mean * rstd
            for j in range(nvec):
                obuf[slot, orow, pl.ds(p * hidden + LANES * j, LANES)] = (
                    (x[j] * rstd + c0) * g[j] + bta[j])

        def chunk_body(c, slot):
            wait_gather(c, slot)

            @pl.when(c >= 2)
            def _():
                wait_scatter(slot)

            base = wid * SEQ_PW * nch + c
            for h in range(SEQ_PW // LANES):
                oidx_v[slot, pl.ds(LANES * h, LANES)] = (
                    base + nch * (LANES * h + lax.iota(jnp.int32, LANES)))

            def egroup(gidx, _):
                f16 = seg_v[c, pl.ds(LANES * gidx, LANES)].astype(jnp.float32)
                for u in range(LANES):
                    segf_v[pl.ds(LANES * (LANES * gidx + u), LANES)] = _permute(
                        f16, jnp.full((LANES,), u, jnp.int32))
                return 0
            lax.fori_loop(0, ch // LANES, egroup, 0)

            for p in range(PBLK):
                pp = PBLK * c + p
                pt0 = [pos_v[pp, pl.ds(LANES * j, LANES)]
                       for j in range(nvec)]

                def srows(si, _):
                    row_norm(slot, p, 2 * si, SEQ_PW * p + 2 * si, pt0)
                    row_norm(slot, p, 2 * si + 1, SEQ_PW * p + 2 * si + 1, pt0)
                    return 0
                lax.fori_loop(0, SEQ_PW // 2, srows, 0)

            issue_scatter(slot)

            @pl.when(c + 2 < nch)
            def _():
                issue_gather(c + 2, slot)

        def outer(gi, _):
            for slot in range(2):
                chunk_body(2 * gi + slot, slot)
            return 0
        lax.fori_loop(0, nch // 2, outer, 0)

        wait_scatter(0)
        wait_scatter(1)

    return sc_kernel


def kernel(tokens, segments, word_emb, pos_emb, type_emb, ln_gamma, ln_beta):
    bsz, seq_len = tokens.shape
    vocab, hidden = word_emb.shape
    n_rows = bsz * seq_len
    def tr(a):
        return (a.reshape(NW, SEQ_PW, seq_len).transpose(0, 2, 1)
                .reshape(NW, seq_len // PBLK, PBLK * SEQ_PW))
    tok = tr(tokens.astype(jnp.int32))
    seg = tr(segments.astype(jnp.int32))
    gb = jnp.stack([ln_gamma, ln_beta]).astype(jnp.float32)
    fn = _make_sc_kernel(n_rows, seq_len, hidden)
    pos0 = (pos_emb[:seq_len] + type_emb[0]).astype(jnp.float32)
    dtv = (type_emb[1] - type_emb[0]).astype(jnp.float32)
    out = fn(tok, seg, word_emb.astype(jnp.float32), pos0, dtv, gb)
    return out.reshape(bsz, seq_len, hidden)


# speedup vs baseline: 1.0195x; 1.0195x over previous
import functools

import jax
import jax.numpy as jnp
from jax import lax
from jax.experimental import pallas as pl
from jax.experimental.pallas import tpu as pltpu
from jax.experimental.pallas import tpu_sc as plsc

NC = 2
NS = 16
NW = NC * NS
LANES = 16
SEQ_PW = 32
HALF = 100
EPS = 1e-5

_DNUMS = lax.GatherDimensionNumbers(
    offset_dims=(), collapsed_slice_dims=(0,), start_index_map=(0,))


def _permute(v, perm):
    return lax.gather(v, perm.reshape(LANES, 1), _DNUMS, (1,),
                      mode=lax.GatherScatterMode.PROMISE_IN_BOUNDS)


def _lane_sum(v):
    for k in (1, 2, 4, 8):
        v = v + _permute(v, lax.iota(jnp.int32, LANES) ^ k)
    return v


def _rsqrt(x):
    i = lax.bitcast_convert_type(x, jnp.int32)
    i = jnp.int32(0x5F3759DF) - lax.shift_right_logical(i, 1)
    y = lax.bitcast_convert_type(i, jnp.float32)
    for _ in range(3):
        y = y * (1.5 - 0.5 * x * y * y)
    return y


def _make_sc_kernel(n_rows, seq_len, hidden):
    nch = SEQ_PW
    nvec = hidden // LANES
    segpad = -(-seq_len // LANES) * LANES
    mesh = plsc.VectorSubcoreMesh(
        core_axis_name="c", subcore_axis_name="s",
        num_cores=NC, num_subcores=NS)

    @functools.partial(
        pl.kernel,
        out_type=jax.ShapeDtypeStruct((n_rows // HALF, HALF, hidden),
                                      jnp.float32),
        mesh=mesh,
        scratch_types=[
            pltpu.VMEM((nch, 2, HALF), jnp.int32),
            pltpu.VMEM((nch, segpad), jnp.int32),
            pltpu.VMEM((seq_len, hidden), jnp.float32),
            pltpu.VMEM((hidden,), jnp.float32),
            pltpu.VMEM((3, 2, HALF, hidden), jnp.float32),
            pltpu.VMEM((2, hidden), jnp.float32),
            pltpu.VMEM((segpad * LANES,), jnp.float32),
            pltpu.SemaphoreType.DMA,
            pltpu.SemaphoreType.DMA,
        ],
    )
    def sc_kernel(tok_hbm, seg_hbm, word_hbm, pos_hbm, dt_hbm, gb_hbm,
                  out_hbm, tok_v, seg_v, pos_v, dt_v, buf, gb_v, segf_v,
                  sem_w, sem_o):
        wid = lax.axis_index("s") * NC + lax.axis_index("c")
        pltpu.sync_copy(tok_hbm.at[wid], tok_v)
        pltpu.sync_copy(seg_hbm.at[wid], seg_v)
        pltpu.sync_copy(gb_hbm, gb_v)
        pltpu.sync_copy(pos_hbm, pos_v)
        pltpu.sync_copy(dt_hbm, dt_v)

        g = [gb_v[0, pl.ds(LANES * j, LANES)] for j in range(nvec)]
        bta = [gb_v[1, pl.ds(LANES * j, LANES)] for j in range(nvec)]
        dt = [dt_v[pl.ds(LANES * j, LANES)] for j in range(nvec)]

        def issue_gathers(c, slot):
            for half in range(2):
                pltpu.async_copy(word_hbm.at[tok_v.at[c, half]],
                                 buf.at[slot, half], sem_w)

        def wait_gathers(c, slot):
            for half in range(2):
                pltpu.make_async_copy(word_hbm.at[tok_v.at[c, half]],
                                      buf.at[slot, half], sem_w).wait()

        def issue_scatter(c, slot):
            blk = (wid * nch + c) * 2
            pltpu.async_copy(buf.at[slot], out_hbm.at[pl.ds(blk, 2)], sem_o)

        def wait_scatter(slot):
            pltpu.make_async_copy(buf.at[slot], out_hbm.at[pl.ds(0, 2)],
                                  sem_o).wait()

        issue_gathers(0, 0)
        issue_gathers(1, 1)

        def row_norm(slot, half, ii, row):
            segf = segf_v[pl.ds(LANES * row, LANES)]
            x = [buf[slot, half, ii, pl.ds(LANES * j, LANES)]
                 + (segf * dt[j] + pos_v[row, pl.ds(LANES * j, LANES)])
                 for j in range(nvec)]
            s = x[0]
            sq = x[0] * x[0]
            for j in range(1, nvec):
                s = s + x[j]
                sq = sq + x[j] * x[j]
            mean = _lane_sum(s) * (1.0 / hidden)
            ex2 = _lane_sum(sq) * (1.0 / hidden)
            var = ex2 - mean * mean
            rstd = _rsqrt(var + EPS)
            c0 = -mean * rstd
            for j in range(nvec):
                buf[slot, half, ii, pl.ds(LANES * j, LANES)] = (
                    (x[j] * rstd + c0) * g[j] + bta[j])

        def chunk_body(c, slot):
            wait_gathers(c, slot)

            def egroup(gidx, _):
                f16 = seg_v[c, pl.ds(LANES * gidx, LANES)].astype(jnp.float32)
                for u in range(LANES):
                    segf_v[pl.ds(LANES * (LANES * gidx + u), LANES)] = (
                        _permute(f16, jnp.full((LANES,), u, jnp.int32)))
                return 0
            lax.fori_loop(0, segpad // LANES, egroup, 0)

            for half in range(2):
                def rows(ii, _):
                    row_norm(slot, half, 2 * ii, HALF * half + 2 * ii)
                    row_norm(slot, half, 2 * ii + 1, HALF * half + 2 * ii + 1)
                    return 0
                lax.fori_loop(0, HALF // 2, rows, 0)

            issue_scatter(c, slot)

            nslot = (slot + 2) % 3

            @pl.when(c >= 1)
            def _():
                wait_scatter(nslot)

            @pl.when(c + 2 < nch)
            def _():
                issue_gathers(c + 2, nslot)

        def outer(gi, _):
            for u in range(3):
                chunk_body(3 * gi + u, u)
            return 0
        lax.fori_loop(0, nch // 3, outer, 0)
        for c in (nch - 2, nch - 1):
            chunk_body(c, c % 3)

        wait_scatter(0)

    return sc_kernel


def kernel(tokens, segments, word_emb, pos_emb, type_emb, ln_gamma, ln_beta):
    bsz, seq_len = tokens.shape
    vocab, hidden = word_emb.shape
    n_rows = bsz * seq_len
    segpad = -(-seq_len // LANES) * LANES
    tok = tokens.reshape(NW, SEQ_PW, 2, HALF).astype(jnp.int32)
    seg = jnp.pad(segments.astype(jnp.int32).reshape(NW, SEQ_PW, seq_len),
                  ((0, 0), (0, 0), (0, segpad - seq_len)))
    gb = jnp.stack([ln_gamma, ln_beta]).astype(jnp.float32)
    pos0 = (pos_emb[:seq_len] + type_emb[0]).astype(jnp.float32)
    dtv = (type_emb[1] - type_emb[0]).astype(jnp.float32)
    fn = _make_sc_kernel(n_rows, seq_len, hidden)
    out = fn(tok, seg, word_emb.astype(jnp.float32), pos0, dtv, gb)
    return out.reshape(bsz, seq_len, hidden)
